# baseline (speedup 1.0000x reference)
import functools

import jax
from jax.experimental import pallas as pl
from jax.experimental.pallas import tpu as pltpu
from jax.experimental.pallas import tpu_sc as plsc

_WIN = 128
_OVERLAP = 0


def kernel(x):
    B, T = x.shape
    stride = _WIN - _OVERLAP
    n_windows = T // _WIN

    mesh = plsc.ScalarSubcoreMesh(axis_name="c", num_cores=1)

    @functools.partial(
        pl.kernel,
        mesh=mesh,
        out_type=jax.ShapeDtypeStruct((B * n_windows * _WIN,), x.dtype),
        scratch_types=[
            pltpu.VMEM_SHARED((B * n_windows * _WIN,), x.dtype),
            pltpu.SemaphoreType.DMA,
            pltpu.SemaphoreType.DMA,
        ],
    )
    def _rolling_window(x_hbm, out_hbm, buf, sem_in, sem_out):
        ins = []
        for b in range(B):
            src = x_hbm.at[b, pl.ds(0, n_windows * stride)]
            dst = buf.at[pl.ds(b * n_windows * _WIN, n_windows * _WIN)]
            ins.append(pltpu.make_async_copy(src, dst, sem_in))
        for c in ins:
            c.start()
        pltpu.make_async_copy(buf, buf, sem_in).wait()
        out_c = pltpu.make_async_copy(buf, out_hbm, sem_out)
        out_c.start()
        out_c.wait()

    out_flat = _rolling_window(x)
    return out_flat.reshape(B, n_windows, _WIN)


# speedup vs baseline: 1.0025x; 1.0025x over previous
import functools

import jax
from jax.experimental import pallas as pl
from jax.experimental.pallas import tpu as pltpu
from jax.experimental.pallas import tpu_sc as plsc

_WIN = 128
_OVERLAP = 0


def kernel(x):
    B, T = x.shape
    stride = _WIN - _OVERLAP
    n_windows = T // _WIN

    mesh = plsc.ScalarSubcoreMesh(axis_name="c", num_cores=1)

    @functools.partial(
        pl.kernel,
        mesh=mesh,
        out_type=jax.ShapeDtypeStruct((B * n_windows * _WIN,), x.dtype),
        scratch_types=[
            pltpu.VMEM_SHARED((B * n_windows * _WIN,), x.dtype),
            pltpu.SemaphoreType.DMA,
        ],
    )
    def _rolling_window(x_hbm, out_hbm, buf, sem):
        sem_in = sem_out = sem
        ins = []
        for b in range(B):
            src = x_hbm.at[b, pl.ds(0, n_windows * stride)]
            dst = buf.at[pl.ds(b * n_windows * _WIN, n_windows * _WIN)]
            ins.append(pltpu.make_async_copy(src, dst, sem_in))
        for c in ins:
            c.start()
        pltpu.make_async_copy(buf, buf, sem_in).wait()
        out_c = pltpu.make_async_copy(buf, out_hbm, sem_out)
        out_c.start()
        out_c.wait()

    out_flat = _rolling_window(x)
    return out_flat.reshape(B, n_windows, _WIN)
